# baseline (speedup 1.0000x reference)
import functools
import jax
import jax.numpy as jnp
from jax import lax
from jax.experimental import pallas as pl
from jax.experimental.pallas import tpu as pltpu
from jax.experimental.pallas import tpu_sc as plsc

_NC, _NS = 2, 16
_NW = _NC * _NS
_CHUNK = 16 * 1024
_NBUF = 3


def _sc_body(x_hbm, pos_hbm, out_hbm, xb0, xb1, xb2, pb0, pb1, pb2,
             xsem, psem, osem):
    xbuf = (xb0, xb1, xb2)
    pbuf = (pb0, pb1, pb2)
    wid = lax.axis_index("s") * _NC + lax.axis_index("c")
    total = x_hbm.shape[0]
    posn = pos_hbm.shape[0]
    per_w = total // _NW
    nch = per_w // _CHUNK
    base = wid * per_w

    def load(ch):
        slot = ch % _NBUF
        off = base + ch * _CHUNK
        poff = lax.rem(off, posn)
        pltpu.async_copy(x_hbm.at[pl.ds(off, _CHUNK)], xbuf[slot],
                         xsem.at[slot])
        pltpu.async_copy(pos_hbm.at[pl.ds(poff, _CHUNK)], pbuf[slot],
                         psem.at[slot])

    for j in range(_NBUF):
        load(j)

    for ch in range(nch):
        slot = ch % _NBUF
        off = base + ch * _CHUNK
        if ch >= 1 and ch + 2 < nch:
            prev_slot = (ch - 1) % _NBUF
            prev_off = base + (ch - 1) * _CHUNK
            pltpu.make_async_copy(
                pbuf[prev_slot], out_hbm.at[pl.ds(prev_off, _CHUNK)],
                osem.at[prev_slot],
            ).wait()
            load(ch + 2)
        pltpu.make_async_copy(
            x_hbm.at[pl.ds(off, _CHUNK)], xbuf[slot], xsem.at[slot]
        ).wait()
        pltpu.make_async_copy(
            pos_hbm.at[pl.ds(lax.rem(off, posn), _CHUNK)], pbuf[slot],
            psem.at[slot],
        ).wait()

        def body(i, carry, slot=slot):
            for k in range(8):
                sl = pl.ds(i * 128 + k * 16, 16)
                plsc.addupdate(pbuf[slot].at[sl], xbuf[slot][sl])
            return carry

        lax.fori_loop(0, _CHUNK // 128, body, 0)
        pltpu.async_copy(pbuf[slot], out_hbm.at[pl.ds(off, _CHUNK)],
                         osem.at[slot])

    for ch in range(nch - 3, nch):
        slot = ch % _NBUF
        off = base + ch * _CHUNK
        pltpu.make_async_copy(
            pbuf[slot], out_hbm.at[pl.ds(off, _CHUNK)], osem.at[slot]
        ).wait()


def kernel(X, pos_embedding):
    B, L, D = X.shape
    mesh = plsc.VectorSubcoreMesh(core_axis_name="c", subcore_axis_name="s")
    run = functools.partial(
        pl.kernel, _sc_body, mesh=mesh,
        out_type=jax.ShapeDtypeStruct((B * L * D,), X.dtype),
        scratch_types=(
            [pltpu.VMEM((_CHUNK,), X.dtype)] * 6
            + [pltpu.SemaphoreType.DMA((_NBUF,))] * 3
        ),
    )()
    out = run(X.reshape(-1), pos_embedding.reshape(-1))
    return out.reshape(B, L, D)


# speedup vs baseline: 5.7691x; 5.7691x over previous
import jax
import jax.numpy as jnp
from jax.experimental import pallas as pl
from jax.experimental.pallas import tpu as pltpu

_C = 1024
_NB = 6


def _pe_kernel(x_hbm, pos_hbm, out_hbm, pos_vmem, xbuf, obuf,
               load_sem, store_sem, pos_sem):
    R = x_hbm.shape[0]
    Lp = pos_hbm.shape[0]
    n = R // _C

    pltpu.make_async_copy(pos_hbm, pos_vmem, pos_sem).start()

    def load(i):
        slot = i % _NB
        pltpu.make_async_copy(
            x_hbm.at[pl.ds(i * _C, _C)], xbuf.at[slot], load_sem.at[slot]
        ).start()

    for j in range(min(_NB, n)):
        load(j)

    pltpu.make_async_copy(pos_hbm, pos_vmem, pos_sem).wait()

    for i in range(n):
        slot = i % _NB
        pltpu.make_async_copy(
            x_hbm.at[pl.ds(i * _C, _C)], xbuf.at[slot], load_sem.at[slot]
        ).wait()
        if i >= _NB:
            pltpu.make_async_copy(
                obuf.at[slot], out_hbm.at[pl.ds((i - _NB) * _C, _C)],
                store_sem.at[slot],
            ).wait()
        off = (i * _C) % Lp
        obuf[slot] = xbuf[slot] + pos_vmem[pl.ds(off, _C)]
        pltpu.make_async_copy(
            obuf.at[slot], out_hbm.at[pl.ds(i * _C, _C)], store_sem.at[slot]
        ).start()
        if i + _NB < n:
            load(i + _NB)

    for i in range(max(0, n - _NB), n):
        slot = i % _NB
        pltpu.make_async_copy(
            obuf.at[slot], out_hbm.at[pl.ds(i * _C, _C)], store_sem.at[slot]
        ).wait()


def kernel(X, pos_embedding):
    B, L, D = X.shape
    out = pl.pallas_call(
        _pe_kernel,
        in_specs=[
            pl.BlockSpec(memory_space=pl.ANY),
            pl.BlockSpec(memory_space=pl.ANY),
        ],
        out_specs=pl.BlockSpec(memory_space=pl.ANY),
        out_shape=jax.ShapeDtypeStruct((B * L, D), X.dtype),
        scratch_shapes=[
            pltpu.VMEM((L, D), X.dtype),
            pltpu.VMEM((_NB, _C, D), X.dtype),
            pltpu.VMEM((_NB, _C, D), X.dtype),
            pltpu.SemaphoreType.DMA((_NB,)),
            pltpu.SemaphoreType.DMA((_NB,)),
            pltpu.SemaphoreType.DMA,
        ],
    )(X.reshape(B * L, D), pos_embedding)
    return out.reshape(B, L, D)


# speedup vs baseline: 5.9142x; 1.0252x over previous
import jax
import jax.numpy as jnp
from jax.experimental import pallas as pl
from jax.experimental.pallas import tpu as pltpu

_C = 2048
_NB = 3


def _pe_kernel(x_hbm, pos_hbm, out_hbm, pos_vmem, xbuf, obuf,
               load_sem, store_sem, pos_sem):
    R = x_hbm.shape[0]
    Lp = pos_hbm.shape[0]
    n = R // _C

    pltpu.make_async_copy(pos_hbm, pos_vmem, pos_sem).start()

    def load(i):
        slot = i % _NB
        pltpu.make_async_copy(
            x_hbm.at[pl.ds(i * _C, _C)], xbuf.at[slot], load_sem.at[slot]
        ).start()

    for j in range(min(_NB, n)):
        load(j)

    pltpu.make_async_copy(pos_hbm, pos_vmem, pos_sem).wait()

    for i in range(n):
        slot = i % _NB
        pltpu.make_async_copy(
            x_hbm.at[pl.ds(i * _C, _C)], xbuf.at[slot], load_sem.at[slot]
        ).wait()
        if i >= _NB:
            pltpu.make_async_copy(
                obuf.at[slot], out_hbm.at[pl.ds((i - _NB) * _C, _C)],
                store_sem.at[slot],
            ).wait()
        off = (i * _C) % Lp
        obuf[slot] = xbuf[slot] + pos_vmem[pl.ds(off, _C)]
        pltpu.make_async_copy(
            obuf.at[slot], out_hbm.at[pl.ds(i * _C, _C)], store_sem.at[slot]
        ).start()
        if i + _NB < n:
            load(i + _NB)

    for i in range(max(0, n - _NB), n):
        slot = i % _NB
        pltpu.make_async_copy(
            obuf.at[slot], out_hbm.at[pl.ds(i * _C, _C)], store_sem.at[slot]
        ).wait()


def kernel(X, pos_embedding):
    B, L, D = X.shape
    out = pl.pallas_call(
        _pe_kernel,
        in_specs=[
            pl.BlockSpec(memory_space=pl.ANY),
            pl.BlockSpec(memory_space=pl.ANY),
        ],
        out_specs=pl.BlockSpec(memory_space=pl.ANY),
        out_shape=jax.ShapeDtypeStruct((B * L, D), X.dtype),
        scratch_shapes=[
            pltpu.VMEM((L, D), X.dtype),
            pltpu.VMEM((_NB, _C, D), X.dtype),
            pltpu.VMEM((_NB, _C, D), X.dtype),
            pltpu.SemaphoreType.DMA((_NB,)),
            pltpu.SemaphoreType.DMA((_NB,)),
            pltpu.SemaphoreType.DMA,
        ],
    )(X.reshape(B * L, D), pos_embedding)
    return out.reshape(B, L, D)
